# baseline (speedup 1.0000x reference)
import jax
import jax.numpy as jnp
from jax import lax
from jax.experimental import pallas as pl
from jax.experimental.pallas import tpu as pltpu
from jax.experimental.pallas import tpu_sc as plsc

_B, _L = 16384, 200
_SUB = 128
_D = 4 * _SUB
_N = _B * _L
_R = 5 + 5 + 128 + 128
_NC, _NS = 2, 16
_NW = _NC * _NS
_PER_W = _N // _NW
_T = 64
_STEPS = _PER_W // _T
_G = 16
_NG = _STEPS // _G


def _sc_body(idx_h, wcat_h, out_h,
             ix0, ix1, buf0, buf1, wcat_v,
             ws0, ws1, isem):
    wid = lax.axis_index("s") * _NC + lax.axis_index("c")
    base = wid * _PER_W
    ix = (ix0, ix1)
    bufs = (buf0, buf1)
    wsem = (ws0, ws1)

    pltpu.sync_copy(wcat_h, wcat_v)

    def fire_idx(g, slot):
        pltpu.make_async_copy(
            idx_h.at[wid, pl.ds(g * _G, _G)], ix[slot], isem).start()

    def wait_idx(slot):
        pltpu.make_async_copy(
            idx_h.at[wid, pl.ds(0, _G)], ix[slot], isem).wait()

    def fire_w(i, slot):
        off = base + i * _T
        pltpu.make_async_copy(
            bufs[slot], out_h.at[pl.ds(off, _T)], wsem[slot]).start()

    def wait_w(slot):
        pltpu.make_async_copy(
            bufs[slot], out_h.at[pl.ds(0, _T)], wsem[slot]).wait()

    def assemble(pos, slot, gslot):
        @plsc.parallel_loop(0, _T // 16)
        def tg_body(tg):
            for j in range(4):
                rowv = ix[gslot][pos, j, pl.ds(tg * 16, 16)]
                for u in range(16):
                    row = rowv[u]
                    t = tg * 16 + u
                    for k in range(8):
                        bufs[slot][t, pl.ds(j * _SUB + k * 16, 16)] = (
                            wcat_v[row, pl.ds(k * 16, 16)])

    fire_idx(0, 0)

    def step(i, carry):
        g = lax.div(i, _G)
        pos = lax.rem(i, _G)
        gslot_i = lax.rem(g, 2)

        @pl.when(pos == 0)
        def _():
            @pl.when(gslot_i == 0)
            def _():
                wait_idx(0)

            @pl.when(gslot_i == 1)
            def _():
                wait_idx(1)

        @pl.when(jnp.logical_and(pos == 1, g + 1 < _NG))
        def _():
            @pl.when(gslot_i == 0)
            def _():
                fire_idx(g + 1, 1)

            @pl.when(gslot_i == 1)
            def _():
                fire_idx(g + 1, 0)

        for slot in range(2):
            @pl.when(lax.rem(i, 2) == slot)
            def _(slot=slot):
                @pl.when(i >= 2)
                def _():
                    wait_w(slot)

                @pl.when(gslot_i == 0)
                def _():
                    assemble(pos, slot, 0)

                @pl.when(gslot_i == 1)
                def _():
                    assemble(pos, slot, 1)

                fire_w(i, slot)

        return carry

    lax.fori_loop(0, _STEPS, step, 0)

    wait_w(_STEPS % 2)
    wait_w((_STEPS + 1) % 2)


@jax.jit
def _run(idx, W_r, W_p, W_r_vel, W_p_vel):
    wcat = jnp.concatenate([W_r, W_p, W_r_vel, W_p_vel], axis=0)

    kern = pl.kernel(
        _sc_body,
        out_type=jax.ShapeDtypeStruct((_N, _D), jnp.float32),
        mesh=plsc.VectorSubcoreMesh(core_axis_name="c", subcore_axis_name="s"),
        scratch_types=[
            pltpu.VMEM((_G, 4, _T), jnp.int32),
            pltpu.VMEM((_G, 4, _T), jnp.int32),
            pltpu.VMEM((_T, _D), jnp.float32),
            pltpu.VMEM((_T, _D), jnp.float32),
            pltpu.VMEM((_R, _SUB), jnp.float32),
            pltpu.SemaphoreType.DMA,
            pltpu.SemaphoreType.DMA,
            pltpu.SemaphoreType.DMA,
        ],
    )
    return kern(idx, wcat)


def kernel(r, p, r_vel, p_vel, W_r, W_p, W_r_vel, W_p_vel):
    r = r.astype(jnp.int32)
    p = p.astype(jnp.int32)
    rv = r_vel.astype(jnp.int32)
    pv = p_vel.astype(jnp.int32)
    idx = jnp.stack([r.reshape(-1), p.reshape(-1) + 5,
                     rv.reshape(-1) + 10, pv.reshape(-1) + 138])
    idx = idx.reshape(4, _NW, _STEPS, _T).transpose(1, 2, 0, 3)
    out = _run(idx, W_r, W_p, W_r_vel, W_p_vel)
    return out.reshape(_B, _L, _D)


# speedup vs baseline: 2.2429x; 2.2429x over previous
import jax
import jax.numpy as jnp
from jax import lax
from jax.experimental import pallas as pl
from jax.experimental.pallas import tpu as pltpu
from jax.experimental.pallas import tpu_sc as plsc

_B, _L = 16384, 200
_SUB = 128
_D = 4 * _SUB
_N = _B * _L
_V = 5 * 5 * 128 * 128
_NC, _NS = 2, 16
_NW = _NC * _NS
_PER_W = _N // _NW
_T = 80
_STEPS = _PER_W // _T
_G = 64
_NG = _STEPS // _G


def _sc_body(idx_h, tab_h, out_h,
             ix0, ix1, buf0, buf1,
             gs0, gs1, ws0, ws1, isem):
    wid = lax.axis_index("s") * _NC + lax.axis_index("c")
    base = wid * _PER_W
    ix = (ix0, ix1)
    bufs = (buf0, buf1)
    gsem = (gs0, gs1)
    wsem = (ws0, ws1)

    def fire_idx(g, slot):
        pltpu.make_async_copy(
            idx_h.at[wid, pl.ds(g * _G, _G)], ix[slot], isem).start()

    def wait_idx(slot):
        pltpu.make_async_copy(
            idx_h.at[wid, pl.ds(0, _G)], ix[slot], isem).wait()

    def fire_g(i, slot, gslot):
        pos = lax.rem(i, _G)
        pltpu.make_async_copy(
            tab_h.at[ix[gslot].at[pos]], bufs[slot], gsem[slot]).start()

    def wait_g(slot, gslot):
        pltpu.make_async_copy(
            tab_h.at[ix[gslot].at[0]], bufs[slot], gsem[slot]).wait()

    def fire_w(i, slot):
        off = base + i * _T
        pltpu.make_async_copy(
            bufs[slot], out_h.at[pl.ds(off, _T)], wsem[slot]).start()

    def wait_w(slot):
        pltpu.make_async_copy(
            bufs[slot], out_h.at[pl.ds(0, _T)], wsem[slot]).wait()

    fire_idx(0, 0)

    def step(i, carry):
        g = lax.div(i, _G)
        pos = lax.rem(i, _G)
        gslot_i = lax.rem(g, 2)

        @pl.when(jnp.logical_and(i < _STEPS, pos == 0))
        def _():
            @pl.when(gslot_i == 0)
            def _():
                wait_idx(0)

            @pl.when(gslot_i == 1)
            def _():
                wait_idx(1)

        @pl.when(jnp.logical_and(pos == 1, g + 1 < _NG))
        def _():
            @pl.when(gslot_i == 0)
            def _():
                fire_idx(g + 1, 1)

            @pl.when(gslot_i == 1)
            def _():
                fire_idx(g + 1, 0)

        @pl.when(i < _STEPS)
        def _():
            for slot in range(2):
                @pl.when(lax.rem(i, 2) == slot)
                def _(slot=slot):
                    @pl.when(i >= 2)
                    def _():
                        wait_w(slot)

                    @pl.when(gslot_i == 0)
                    def _():
                        fire_g(i, slot, 0)

                    @pl.when(gslot_i == 1)
                    def _():
                        fire_g(i, slot, 1)

        @pl.when(i >= 1)
        def _():
            ip = i - 1
            gslot_p = lax.rem(lax.div(ip, _G), 2)
            for slot in range(2):
                @pl.when(lax.rem(ip, 2) == slot)
                def _(slot=slot):
                    @pl.when(gslot_p == 0)
                    def _():
                        wait_g(slot, 0)

                    @pl.when(gslot_p == 1)
                    def _():
                        wait_g(slot, 1)

                    fire_w(ip, slot)

        return carry

    lax.fori_loop(0, _STEPS + 1, step, 0)

    wait_w(_STEPS % 2)
    wait_w((_STEPS + 1) % 2)


@jax.jit
def _run(idx, W_r, W_p, W_r_vel, W_p_vel):
    shape5 = (5, 5, 128, 128, _SUB)
    tab = jnp.concatenate([
        jnp.broadcast_to(W_r[:, None, None, None, :], shape5),
        jnp.broadcast_to(W_p[None, :, None, None, :], shape5),
        jnp.broadcast_to(W_r_vel[None, None, :, None, :], shape5),
        jnp.broadcast_to(W_p_vel[None, None, None, :, :], shape5),
    ], axis=-1).reshape(_V, _D)

    kern = pl.kernel(
        _sc_body,
        out_type=jax.ShapeDtypeStruct((_N, _D), jnp.float32),
        mesh=plsc.VectorSubcoreMesh(core_axis_name="c", subcore_axis_name="s"),
        scratch_types=[
            pltpu.VMEM((_G, _T), jnp.int32),
            pltpu.VMEM((_G, _T), jnp.int32),
            pltpu.VMEM((_T, _D), jnp.float32),
            pltpu.VMEM((_T, _D), jnp.float32),
            pltpu.SemaphoreType.DMA,
            pltpu.SemaphoreType.DMA,
            pltpu.SemaphoreType.DMA,
            pltpu.SemaphoreType.DMA,
            pltpu.SemaphoreType.DMA,
        ],
    )
    return kern(idx, tab)


def kernel(r, p, r_vel, p_vel, W_r, W_p, W_r_vel, W_p_vel):
    r = r.astype(jnp.int32)
    p = p.astype(jnp.int32)
    rv = r_vel.astype(jnp.int32)
    pv = p_vel.astype(jnp.int32)
    idx = (((r * 5 + p) * 128 + rv) * 128 + pv).reshape(_NW, _STEPS, _T)
    out = _run(idx, W_r, W_p, W_r_vel, W_p_vel)
    return out.reshape(_B, _L, _D)
